# baseline (device time: 39726 ns/iter reference)
import jax
import jax.numpy as jnp
from jax import lax
from jax.experimental import pallas as pl
from jax.experimental.pallas import tpu as pltpu

N_DEV = 4
B, Sq, Skv, Hq_G, Dh = 2, 512, 512, 32, 64
H_LOC = Hq_G // N_DEV
DQK = H_LOC * Dh
DM = 768
BLK = 64
N_CHUNK = 2
H_CH = H_LOC // N_CHUNK
W_CH = H_CH * Dh


def kernel(x, Wq, K_ext, V_ext, Wo):
    my = lax.axis_index("i")
    Wq_loc = lax.dynamic_slice_in_dim(Wq, my * DQK, DQK, axis=1)

    def body(x_ref, wq_ref, k_ref, v_ref, wo_ref, out_ref,
             buf_ref, send_sems, recv_sems):
        my_pos = lax.axis_index("i")

        barrier_sem = pltpu.get_barrier_semaphore()
        for d in range(1, N_DEV):
            pl.semaphore_signal(
                barrier_sem, inc=1,
                device_id=(lax.rem(my_pos + d, N_DEV),),
                device_id_type=pl.DeviceIdType.MESH,
            )
        pl.semaphore_wait(barrier_sem, N_DEV - 1)

        rdma = [
            [
                [
                    pltpu.make_async_remote_copy(
                        src_ref=buf_ref.at[0, b, :, c * W_CH:(c + 1) * W_CH],
                        dst_ref=buf_ref.at[d, b, :, c * W_CH:(c + 1) * W_CH],
                        send_sem=send_sems.at[d - 1, b, c],
                        recv_sem=recv_sems.at[d - 1, b, c],
                        device_id=(lax.rem(my_pos + d, N_DEV),),
                        device_id_type=pl.DeviceIdType.MESH,
                    )
                    for c in range(N_CHUNK)
                ]
                for b in range(B)
            ]
            for d in range(1, N_DEV)
        ]

        qb = lax.broadcasted_iota(jnp.int32, (Sq, Skv), 0) // BLK
        kb = lax.broadcasted_iota(jnp.int32, (Sq, Skv), 1) // BLK
        bias = jnp.where((qb % 4) == (kb % 4), 0.0, -1e9).astype(jnp.float32)
        ones_col = jnp.ones((Skv, 1), jnp.bfloat16)

        wq_bf = wq_ref[:, :].astype(jnp.bfloat16)
        q_cache = {}

        def attn_chunk(b, c):
            if b not in q_cache:
                xb = x_ref[b, :, :].astype(jnp.bfloat16)
                q = jnp.dot(xb, wq_bf, preferred_element_type=jnp.float32)
                q_cache[b] = (q * 0.125).astype(jnp.bfloat16)
            q_all = q_cache[b]
            for j in range(H_CH):
                h = c * H_CH + j
                q_h = q_all[:, h * Dh:(h + 1) * Dh]
                k_h = k_ref[b, :, h, :].astype(jnp.bfloat16)
                s = lax.dot_general(
                    q_h, k_h, (((1,), (1,)), ((), ())),
                    preferred_element_type=jnp.float32,
                )
                p = jnp.exp(s + bias).astype(jnp.bfloat16)
                v_h = v_ref[b, :, h, :].astype(jnp.bfloat16)
                v_aug = jnp.concatenate([v_h, ones_col], axis=1)
                ctx_ext = jnp.dot(p, v_aug,
                                  preferred_element_type=jnp.float32)
                ctx = ctx_ext[:, :Dh] * (1.0 / ctx_ext[:, Dh:Dh + 1])
                buf_ref[0, b, :, h * Dh:(h + 1) * Dh] = ctx.astype(jnp.bfloat16)

        def fold(b, d):
            origin = lax.rem(my_pos + N_DEV - d, N_DEV)
            wo_rows = wo_ref[pl.ds(origin * DQK, DQK), :]
            out_ref[b, :, :] += jnp.dot(buf_ref[d, b],
                                        wo_rows.astype(jnp.bfloat16),
                                        preferred_element_type=jnp.float32)

        for b in range(B):
            for c in range(N_CHUNK):
                attn_chunk(b, c)
                for d in (1, 3, 2):
                    rdma[d - 1][b][c].start()

        out_ref[:, :, :] = jnp.zeros((B, Sq, DM), jnp.float32)
        for b in range(B):
            fold(b, 0)

        for b in range(B):
            for d in (1, 3, 2):
                for c in range(N_CHUNK):
                    rdma[d - 1][b][c].wait_recv()
                fold(b, d)

        for d in range(1, N_DEV):
            for b in range(B):
                for c in range(N_CHUNK):
                    rdma[d - 1][b][c].wait_send()

    return pl.pallas_call(
        body,
        out_shape=jax.ShapeDtypeStruct((B, Sq, DM), jnp.float32),
        in_specs=[pl.BlockSpec(memory_space=pltpu.VMEM)] * 5,
        out_specs=pl.BlockSpec(memory_space=pltpu.VMEM),
        scratch_shapes=[
            pltpu.VMEM((N_DEV, B, Sq, DQK), jnp.bfloat16),
            pltpu.SemaphoreType.DMA((N_DEV - 1, B, N_CHUNK)),
            pltpu.SemaphoreType.DMA((N_DEV - 1, B, N_CHUNK)),
        ],
        compiler_params=pltpu.CompilerParams(collective_id=0),
    )(x, Wq_loc, K_ext, V_ext, Wo)


# device time: 27457 ns/iter; 1.4468x vs baseline; 1.4468x over previous
import jax
import jax.numpy as jnp
from jax import lax
from jax.experimental import pallas as pl
from jax.experimental.pallas import tpu as pltpu

N_DEV = 4
B, Sq, Skv, Hq_G, Dh = 2, 512, 512, 32, 64
H_LOC = Hq_G // N_DEV
DQK = H_LOC * Dh
DM = 768
BLK = 64
N_CHUNK = 2
H_CH = H_LOC // N_CHUNK
W_CH = H_CH * Dh


def kernel(x, Wq, K_ext, V_ext, Wo):
    my = lax.axis_index("i")
    Wq_loc = lax.dynamic_slice_in_dim(Wq, my * DQK, DQK, axis=1)
    KT = jnp.transpose(K_ext.astype(jnp.bfloat16), (0, 2, 3, 1))
    VT = jnp.transpose(V_ext.astype(jnp.bfloat16), (0, 2, 1, 3))

    def body(x_ref, wq_ref, k_ref, v_ref, wo_ref, out_ref,
             buf_ref, scl_ref, send_sems, recv_sems, s2_send, s2_recv):
        my_pos = lax.axis_index("i")

        barrier_sem = pltpu.get_barrier_semaphore()
        for d in range(1, N_DEV):
            pl.semaphore_signal(
                barrier_sem, inc=1,
                device_id=(lax.rem(my_pos + d, N_DEV),),
                device_id_type=pl.DeviceIdType.MESH,
            )
        pl.semaphore_wait(barrier_sem, N_DEV - 1)

        rdma = [
            [
                [
                    pltpu.make_async_remote_copy(
                        src_ref=buf_ref.at[0, b, :, c * W_CH:(c + 1) * W_CH],
                        dst_ref=buf_ref.at[d, b, :, c * W_CH:(c + 1) * W_CH],
                        send_sem=send_sems.at[d - 1, b, c],
                        recv_sem=recv_sems.at[d - 1, b, c],
                        device_id=(lax.rem(my_pos + d, N_DEV),),
                        device_id_type=pl.DeviceIdType.MESH,
                    )
                    for c in range(N_CHUNK)
                ]
                for b in range(B)
            ]
            for d in range(1, N_DEV)
        ]
        rdma_s = [
            [
                [
                    pltpu.make_async_remote_copy(
                        src_ref=scl_ref.at[0, b, c],
                        dst_ref=scl_ref.at[d, b, c],
                        send_sem=s2_send.at[d - 1, b, c],
                        recv_sem=s2_recv.at[d - 1, b, c],
                        device_id=(lax.rem(my_pos + d, N_DEV),),
                        device_id_type=pl.DeviceIdType.MESH,
                    )
                    for c in range(N_CHUNK)
                ]
                for b in range(B)
            ]
            for d in range(1, N_DEV)
        ]

        qb = lax.broadcasted_iota(jnp.int32, (Sq, Skv), 0) // BLK
        kb = lax.broadcasted_iota(jnp.int32, (Sq, Skv), 1) // BLK
        bias = jnp.where((qb % 4) == (kb % 4), 0.0, -1e9).astype(jnp.float32)
        ones_col = jnp.ones((Skv, 1), jnp.bfloat16)

        wq_bf = wq_ref[:, :].astype(jnp.bfloat16)
        q_cache = {}

        def attn_chunk(b, c):
            if b not in q_cache:
                xb = x_ref[b, :, :].astype(jnp.bfloat16)
                q = jnp.dot(xb, wq_bf, preferred_element_type=jnp.float32)
                q_cache[b] = (q * 0.125).astype(jnp.bfloat16)
            q_all = q_cache[b]
            ctxs = []
            for j in range(H_CH):
                h = c * H_CH + j
                q_h = q_all[:, h * Dh:(h + 1) * Dh]
                k_t = k_ref[b, h]
                s = jnp.dot(q_h, k_t,
                            preferred_element_type=jnp.float32)
                p = jnp.exp(s + bias).astype(jnp.bfloat16)
                v_h = v_ref[b, h]
                v_aug = jnp.concatenate([v_h, ones_col], axis=1)
                ctx_ext = jnp.dot(p, v_aug,
                                  preferred_element_type=jnp.float32)
                ctxs.append(ctx_ext[:, :Dh] * (1.0 / ctx_ext[:, Dh:Dh + 1]))
            a = jnp.concatenate(ctxs, axis=1)
            m = jnp.max(jnp.abs(a), axis=0, keepdims=True)
            m = jnp.maximum(m, 1e-20)
            buf_ref[0, b, :, c * W_CH:(c + 1) * W_CH] = jnp.round(
                a * (127.0 / m)).astype(jnp.int8)
            scl_ref[0, b, c] = m * (1.0 / 127.0)

        def fold(b, d, first=False):
            origin = lax.rem(my_pos + N_DEV - d, N_DEV)
            dq = jnp.concatenate(
                [
                    (buf_ref[d, b, :, c * W_CH:(c + 1) * W_CH].astype(
                        jnp.float32) * scl_ref[d, b, c]).astype(jnp.bfloat16)
                    for c in range(N_CHUNK)
                ],
                axis=1,
            )
            wo_rows = wo_ref[pl.ds(origin * DQK, DQK), :]
            r = jnp.dot(dq, wo_rows.astype(jnp.bfloat16),
                        preferred_element_type=jnp.float32)
            if first:
                out_ref[b, :, :] = r
            else:
                out_ref[b, :, :] += r

        for b in range(B):
            for c in range(N_CHUNK):
                attn_chunk(b, c)
                for d in (1, 3, 2):
                    rdma[d - 1][b][c].start()
                    rdma_s[d - 1][b][c].start()

        for b in range(B):
            fold(b, 0, first=True)

        for b in range(B):
            for d in (1, 3, 2):
                for c in range(N_CHUNK):
                    rdma[d - 1][b][c].wait_recv()
                    rdma_s[d - 1][b][c].wait_recv()
                fold(b, d)

        for d in range(1, N_DEV):
            for b in range(B):
                for c in range(N_CHUNK):
                    rdma[d - 1][b][c].wait_send()
                    rdma_s[d - 1][b][c].wait_send()

    return pl.pallas_call(
        body,
        out_shape=jax.ShapeDtypeStruct((B, Sq, DM), jnp.float32),
        in_specs=[pl.BlockSpec(memory_space=pltpu.VMEM)] * 5,
        out_specs=pl.BlockSpec(memory_space=pltpu.VMEM),
        scratch_shapes=[
            pltpu.VMEM((N_DEV, B, Sq, DQK), jnp.int8),
            pltpu.VMEM((N_DEV, B, N_CHUNK, 1, W_CH), jnp.float32),
            pltpu.SemaphoreType.DMA((N_DEV - 1, B, N_CHUNK)),
            pltpu.SemaphoreType.DMA((N_DEV - 1, B, N_CHUNK)),
            pltpu.SemaphoreType.DMA((N_DEV - 1, B, N_CHUNK)),
            pltpu.SemaphoreType.DMA((N_DEV - 1, B, N_CHUNK)),
        ],
        compiler_params=pltpu.CompilerParams(collective_id=0),
    )(x, Wq_loc, KT, VT, Wo)
